# baseline (speedup 1.0000x reference)
import functools
import jax
import jax.numpy as jnp
from jax import lax
from jax.experimental import pallas as pl
from jax.experimental.pallas import tpu as pltpu, tpu_sc as plsc

F = 100
BINS = 100
PADB = 128
BLK = 2048
PAT = 400
NV = PAT // 16
CH = 12800
NW = 32

SPLIT_TC = 0


def _tables_body(lt_ref, lu_ref, smin_ref, smax_ref,
                 ds_ref, l_ref, c0_ref, e_ref, a_ref, c_ref):
    lt = lt_ref[...]
    lu = lu_ref[...]
    smin = smin_ref[...]
    smax = smax_ref[...]
    wu = jnp.exp(lu) + 1e-7
    wt = jax.nn.softmax(lt, axis=1) + 1e-7
    wt = wt / jnp.sum(wt, axis=1, keepdims=True)
    s = wu / wt
    row = lax.broadcasted_iota(jnp.int32, (BINS, BINS), 0)
    col = lax.broadcasted_iota(jnp.int32, (BINS, BINS), 1)
    tri_strict = (row < col).astype(jnp.float32)
    tri_incl = (row <= col).astype(jnp.float32)
    lt_left = jnp.dot(wt, tri_strict, preferred_element_type=jnp.float32)
    lu_left = jnp.dot(wu, tri_strict, preferred_element_type=jnp.float32)
    et_right = jnp.dot(wt, tri_incl, preferred_element_type=jnp.float32)
    rng = smax - smin
    invr = 1.0 / rng
    ds = s - jnp.concatenate([jnp.zeros((F, 1), jnp.float32), s[:, :-1]], axis=1)
    ds_scaled = ds * invr
    l_scaled = smin + lt_left * rng
    ds_ref[...] = ds_scaled
    l_ref[...] = l_scaled
    c0_ref[...] = jnp.sum(ds_scaled * l_scaled, axis=1, keepdims=True)
    a = s * invr
    e_ref[...] = smin + et_right * rng
    a_ref[...] = a
    c_ref[...] = lu_left - s * lt_left - a * smin


def _build_tables(logits_t, logits_u, sigma_min, sigma_max):
    smin = sigma_min.reshape(F, 1)
    smax = sigma_max.reshape(F, 1)
    fb = jax.ShapeDtypeStruct((F, BINS), jnp.float32)
    return pl.pallas_call(
        _tables_body,
        out_shape=(fb, fb, jax.ShapeDtypeStruct((F, 1), jnp.float32),
                   fb, fb, fb),
    )(logits_t, logits_u, smin, smax)


def _sweep_body(x_ref, ds_ref, l_ref, c0_ref, o_ref):
    xb = x_ref[...]
    acc = jnp.zeros_like(xb)
    for k in range(BINS):
        acc = acc + ds_ref[k:k + 1, :] * jnp.maximum(xb, l_ref[k:k + 1, :])
    o_ref[...] = acc - c0_ref[0:1, :]


def _tc_sweep(x, ds, l, c0):
    b = x.shape[0]
    return pl.pallas_call(
        _sweep_body,
        grid=(b // BLK,),
        in_specs=[
            pl.BlockSpec((BLK, F), lambda i: (i, 0)),
            pl.BlockSpec((BINS, F), lambda i: (0, 0)),
            pl.BlockSpec((BINS, F), lambda i: (0, 0)),
            pl.BlockSpec((1, F), lambda i: (0, 0)),
        ],
        out_specs=pl.BlockSpec((BLK, F), lambda i: (i, 0)),
        out_shape=jax.ShapeDtypeStruct((b, F), jnp.float32),
    )(x, ds.T, l.T, c0.T)


def _sc_sweep(x_flat, e_pad, a_pad, c_pad, base_pat, base99_pat):
    n = x_flat.shape[0]
    per_w = n // NW
    nch = per_w // CH
    groups = CH // PAT
    mesh = plsc.VectorSubcoreMesh(core_axis_name="c", subcore_axis_name="s")

    @functools.partial(
        pl.kernel, mesh=mesh,
        compiler_params=pltpu.CompilerParams(needs_layout_passes=False),
        out_type=jax.ShapeDtypeStruct((n,), jnp.float32),
        scratch_types=[
            pltpu.VMEM((CH,), jnp.float32),
            pltpu.VMEM((CH,), jnp.float32),
            pltpu.VMEM((F * PADB,), jnp.float32),
            pltpu.VMEM((F * PADB,), jnp.float32),
            pltpu.VMEM((F * PADB,), jnp.float32),
            pltpu.VMEM((PAT,), jnp.int32),
            pltpu.VMEM((PAT,), jnp.int32),
        ],
    )
    def body(x_hbm, e_hbm, a_hbm, c_hbm, pb_hbm, p99_hbm, out_hbm,
             xbuf, obuf, ebuf, abuf, cbuf, pbbuf, p99buf):
        wid = lax.axis_index("s") * 2 + lax.axis_index("c")
        pltpu.sync_copy(e_hbm, ebuf)
        pltpu.sync_copy(a_hbm, abuf)
        pltpu.sync_copy(c_hbm, cbuf)
        pltpu.sync_copy(pb_hbm, pbbuf)
        pltpu.sync_copy(p99_hbm, p99buf)
        w_base = wid * per_w

        def chunk_body(ci, carry):
            start = w_base + ci * CH
            pltpu.sync_copy(x_hbm.at[pl.ds(start, CH)], xbuf)

            def group_body(g, carry2):
                goff = g * PAT
                for v in range(NV):
                    off = goff + v * 16
                    xv = xbuf[pl.ds(off, 16)]
                    jb = pbbuf[pl.ds(v * 16, 16)]
                    for step in (64, 32, 16, 8, 4, 2, 1):
                        probe = jb + (step - 1)
                        ev = plsc.load_gather(ebuf, [probe])
                        jb = jnp.where(ev < xv, jb + step, jb)
                    jc = jnp.minimum(jb, p99buf[pl.ds(v * 16, 16)])
                    av = plsc.load_gather(abuf, [jc])
                    cv = plsc.load_gather(cbuf, [jc])
                    obuf[pl.ds(off, 16)] = av * xv + cv
                return carry2

            lax.fori_loop(0, groups, group_body, 0)
            pltpu.sync_copy(obuf, out_hbm.at[pl.ds(start, CH)])
            return carry

        lax.fori_loop(0, nch, chunk_body, 0)

    return body(x_flat, e_pad, a_pad, c_pad, base_pat, base99_pat)


@jax.jit
def kernel(x, logits_t, logits_u, sigma_min, sigma_max):
    ds, l, c0, e, a, c = _build_tables(logits_t, logits_u, sigma_min, sigma_max)
    b = x.shape[0]
    padw = ((0, 0), (0, PADB - BINS))
    e_pad = jnp.pad(e, padw, constant_values=jnp.inf).reshape(-1)
    a_pad = jnp.pad(a, padw).reshape(-1)
    c_pad = jnp.pad(c, padw).reshape(-1)
    fid = (jnp.arange(PAT, dtype=jnp.int32) % F) * PADB
    outs = []
    if SPLIT_TC > 0:
        outs.append(_tc_sweep(x[:SPLIT_TC], ds, l, c0))
    if SPLIT_TC < b:
        x_flat = x[SPLIT_TC:].reshape(-1)
        o = _sc_sweep(x_flat, e_pad, a_pad, c_pad, fid, fid + 99)
        outs.append(o.reshape(b - SPLIT_TC, F))
    return outs[0] if len(outs) == 1 else jnp.concatenate(outs, axis=0)


# speedup vs baseline: 1.7997x; 1.7997x over previous
import functools
import jax
import jax.numpy as jnp
from jax import lax
from jax.experimental import pallas as pl
from jax.experimental.pallas import tpu as pltpu, tpu_sc as plsc

F = 100
BINS = 100
PADB = 128
BLK = 2048
PAT = 400
NV = PAT // 16
NCHAIN = 5
CH = 12800
NW = 32

SPLIT_TC = 0


def _tables_body(lt_ref, lu_ref, smin_ref, smax_ref,
                 ds_ref, l_ref, c0_ref, e_ref, a_ref, c_ref):
    lt = lt_ref[...]
    lu = lu_ref[...]
    smin = smin_ref[...]
    smax = smax_ref[...]
    wu = jnp.exp(lu) + 1e-7
    wt = jax.nn.softmax(lt, axis=1) + 1e-7
    wt = wt / jnp.sum(wt, axis=1, keepdims=True)
    s = wu / wt
    row = lax.broadcasted_iota(jnp.int32, (BINS, BINS), 0)
    col = lax.broadcasted_iota(jnp.int32, (BINS, BINS), 1)
    tri_strict = (row < col).astype(jnp.float32)
    tri_incl = (row <= col).astype(jnp.float32)
    lt_left = jnp.dot(wt, tri_strict, preferred_element_type=jnp.float32)
    lu_left = jnp.dot(wu, tri_strict, preferred_element_type=jnp.float32)
    et_right = jnp.dot(wt, tri_incl, preferred_element_type=jnp.float32)
    rng = smax - smin
    invr = 1.0 / rng
    ds = s - jnp.concatenate([jnp.zeros((F, 1), jnp.float32), s[:, :-1]], axis=1)
    ds_scaled = ds * invr
    l_scaled = smin + lt_left * rng
    ds_ref[...] = ds_scaled
    l_ref[...] = l_scaled
    c0_ref[...] = jnp.sum(ds_scaled * l_scaled, axis=1, keepdims=True)
    a = s * invr
    e_ref[...] = smin + et_right * rng
    a_ref[...] = a
    c_ref[...] = lu_left - s * lt_left - a * smin


def _build_tables(logits_t, logits_u, sigma_min, sigma_max):
    smin = sigma_min.reshape(F, 1)
    smax = sigma_max.reshape(F, 1)
    fb = jax.ShapeDtypeStruct((F, BINS), jnp.float32)
    return pl.pallas_call(
        _tables_body,
        out_shape=(fb, fb, jax.ShapeDtypeStruct((F, 1), jnp.float32),
                   fb, fb, fb),
    )(logits_t, logits_u, smin, smax)


def _sweep_body(x_ref, ds_ref, l_ref, c0_ref, o_ref):
    xb = x_ref[...]
    acc = jnp.zeros_like(xb)
    for k in range(BINS):
        acc = acc + ds_ref[k:k + 1, :] * jnp.maximum(xb, l_ref[k:k + 1, :])
    o_ref[...] = acc - c0_ref[0:1, :]


def _tc_sweep(x, ds, l, c0):
    b = x.shape[0]
    return pl.pallas_call(
        _sweep_body,
        grid=(b // BLK,),
        in_specs=[
            pl.BlockSpec((BLK, F), lambda i: (i, 0)),
            pl.BlockSpec((BINS, F), lambda i: (0, 0)),
            pl.BlockSpec((BINS, F), lambda i: (0, 0)),
            pl.BlockSpec((1, F), lambda i: (0, 0)),
        ],
        out_specs=pl.BlockSpec((BLK, F), lambda i: (i, 0)),
        out_shape=jax.ShapeDtypeStruct((b, F), jnp.float32),
    )(x, ds.T, l.T, c0.T)


def _sc_sweep(x_flat, e_pad, a_pad, c_pad, base_pat, base99_pat):
    n = x_flat.shape[0]
    per_w = n // NW
    nch = per_w // CH
    groups = CH // PAT
    mesh = plsc.VectorSubcoreMesh(core_axis_name="c", subcore_axis_name="s")

    @functools.partial(
        pl.kernel, mesh=mesh,
        compiler_params=pltpu.CompilerParams(needs_layout_passes=False),
        out_type=jax.ShapeDtypeStruct((n,), jnp.float32),
        scratch_types=[
            pltpu.VMEM((CH,), jnp.float32),
            pltpu.VMEM((CH,), jnp.float32),
            pltpu.VMEM((F * PADB,), jnp.float32),
            pltpu.VMEM((F * PADB,), jnp.float32),
            pltpu.VMEM((F * PADB,), jnp.float32),
            pltpu.VMEM((PAT,), jnp.int32),
            pltpu.VMEM((PAT,), jnp.int32),
        ],
    )
    def body(x_hbm, e_hbm, a_hbm, c_hbm, pb_hbm, p99_hbm, out_hbm,
             xbuf, obuf, ebuf, abuf, cbuf, pbbuf, p99buf):
        wid = lax.axis_index("s") * 2 + lax.axis_index("c")
        pltpu.sync_copy(e_hbm, ebuf)
        pltpu.sync_copy(a_hbm, abuf)
        pltpu.sync_copy(c_hbm, cbuf)
        pltpu.sync_copy(pb_hbm, pbbuf)
        pltpu.sync_copy(p99_hbm, p99buf)
        w_base = wid * per_w

        def chunk_body(ci, carry):
            start = w_base + ci * CH
            pltpu.sync_copy(x_hbm.at[pl.ds(start, CH)], xbuf)

            def group_body(g, carry2):
                goff = g * PAT
                for c in range(NV // NCHAIN):
                    vs = [c * NCHAIN + u for u in range(NCHAIN)]
                    xs = [xbuf[pl.ds(goff + v * 16, 16)] for v in vs]
                    jbs = [pbbuf[pl.ds(v * 16, 16)] for v in vs]
                    for step in (64, 32, 16, 8, 4, 2, 1):
                        evs = [plsc.load_gather(ebuf, [jb + (step - 1)])
                               for jb in jbs]
                        jbs = [jnp.where(ev < xv, jb + step, jb)
                               for ev, xv, jb in zip(evs, xs, jbs)]
                    jcs = [jnp.minimum(jb, p99buf[pl.ds(v * 16, 16)])
                           for jb, v in zip(jbs, vs)]
                    avs = [plsc.load_gather(abuf, [jc]) for jc in jcs]
                    cvs = [plsc.load_gather(cbuf, [jc]) for jc in jcs]
                    for xv, av, cv, v in zip(xs, avs, cvs, vs):
                        obuf[pl.ds(goff + v * 16, 16)] = av * xv + cv
                return carry2

            lax.fori_loop(0, groups, group_body, 0)
            pltpu.sync_copy(obuf, out_hbm.at[pl.ds(start, CH)])
            return carry

        lax.fori_loop(0, nch, chunk_body, 0)

    return body(x_flat, e_pad, a_pad, c_pad, base_pat, base99_pat)


@jax.jit
def kernel(x, logits_t, logits_u, sigma_min, sigma_max):
    ds, l, c0, e, a, c = _build_tables(logits_t, logits_u, sigma_min, sigma_max)
    b = x.shape[0]
    padw = ((0, 0), (0, PADB - BINS))
    e_pad = jnp.pad(e, padw, constant_values=jnp.inf).reshape(-1)
    a_pad = jnp.pad(a, padw).reshape(-1)
    c_pad = jnp.pad(c, padw).reshape(-1)
    fid = (jnp.arange(PAT, dtype=jnp.int32) % F) * PADB
    outs = []
    if SPLIT_TC > 0:
        outs.append(_tc_sweep(x[:SPLIT_TC], ds, l, c0))
    if SPLIT_TC < b:
        x_flat = x[SPLIT_TC:].reshape(-1)
        o = _sc_sweep(x_flat, e_pad, a_pad, c_pad, fid, fid + 99)
        outs.append(o.reshape(b - SPLIT_TC, F))
    return outs[0] if len(outs) == 1 else jnp.concatenate(outs, axis=0)
